# baseline (speedup 1.0000x reference)
import functools

import jax
import jax.numpy as jnp
from jax import lax
from jax.experimental import pallas as pl
from jax.experimental.pallas import tpu as pltpu
from jax.experimental.pallas import tpu_sc as plsc


_ROWS_PER_DMA = 128


def _sc_gather(word_emb, tok_flat, rows_per_dma):
    D = word_emb.shape[1]
    info = plsc.get_sparse_core_info()
    NC, NS = info.num_cores, info.num_subcores
    NW = NC * NS
    N = tok_flat.shape[0]
    per_w = N // NW
    n_chunks = per_w // rows_per_dma
    mesh = plsc.VectorSubcoreMesh(core_axis_name="c", subcore_axis_name="s")

    @functools.partial(
        pl.kernel,
        out_type=jax.ShapeDtypeStruct((N, D), jnp.float32),
        mesh=mesh,
        scratch_types=[
            pltpu.VMEM((per_w,), jnp.int32),
            pltpu.VMEM((rows_per_dma, D), jnp.float32),
            pltpu.VMEM((rows_per_dma, D), jnp.float32),
            pltpu.SemaphoreType.DMA,
            pltpu.SemaphoreType.DMA,
        ],
    )
    def k(table_hbm, idx_hbm, out_hbm, idx_v, rows0, rows1, sem0, sem1):
        wid = lax.axis_index("s") * NC + lax.axis_index("c")
        base = wid * per_w
        pltpu.sync_copy(idx_hbm.at[pl.ds(base, per_w)], idx_v)
        rows = (rows0, rows1)
        sems = (sem0, sem1)
        pltpu.async_copy(
            table_hbm.at[idx_v.at[pl.ds(0, rows_per_dma)]], rows0, sem0)

        def chunk(j, _):
            @pl.when(j + 1 < n_chunks)
            def _():
                for b in range(2):
                    @pl.when(lax.rem(j + 1, 2) == b)
                    def _():
                        pltpu.async_copy(
                            table_hbm.at[
                                idx_v.at[pl.ds((j + 1) * rows_per_dma,
                                               rows_per_dma)]],
                            rows[b], sems[b])

            for b in range(2):
                @pl.when(lax.rem(j, 2) == b)
                def _():
                    pltpu.make_async_copy(
                        table_hbm.at[
                            idx_v.at[pl.ds(j * rows_per_dma,
                                           rows_per_dma)]],
                        rows[b], sems[b]).wait()
                    pltpu.sync_copy(
                        rows[b],
                        out_hbm.at[pl.ds(base + j * rows_per_dma,
                                         rows_per_dma)])
            return 0

        lax.fori_loop(0, n_chunks, chunk, 0)

    return k(word_emb, tok_flat)


_EPS = 1e-12


def _ln_body(g_ref, seg_ref, pe_ref, te_ref, gamma_ref, beta_ref, o_ref):
    pe = pe_ref[...]
    dte = te_ref[0, :]
    segf = seg_ref[...].astype(jnp.float32)
    x = g_ref[...] + pe[None, :, :] + segf[..., None] * dte
    mu = jnp.mean(x, axis=-1, keepdims=True)
    m2 = jnp.mean(x * x, axis=-1, keepdims=True)
    r = lax.rsqrt(m2 - mu * mu + _EPS)
    o_ref[...] = (x - mu) * r * gamma_ref[0, :] + beta_ref[0, :]


def _tc_ln(gathered, segment_ids, pe, te, gamma, beta, b_blk):
    B, S, D = gathered.shape
    grid = (B // b_blk,)
    return pl.pallas_call(
        _ln_body,
        grid=grid,
        in_specs=[
            pl.BlockSpec((b_blk, S, D), lambda i: (i, 0, 0)),
            pl.BlockSpec((b_blk, S), lambda i: (i, 0)),
            pl.BlockSpec((S, D), lambda i: (0, 0)),
            pl.BlockSpec((1, D), lambda i: (0, 0)),
            pl.BlockSpec((1, D), lambda i: (0, 0)),
            pl.BlockSpec((1, D), lambda i: (0, 0)),
        ],
        out_specs=pl.BlockSpec((b_blk, S, D), lambda i: (i, 0, 0)),
        out_shape=jax.ShapeDtypeStruct((B, S, D), jnp.float32),
    )(gathered, segment_ids, pe, te, gamma, beta)


def _sc_fused(word_emb, tok_flat, seg_flat, pe0, dte, gamma, beta, S):
    D = word_emb.shape[1]
    info = plsc.get_sparse_core_info()
    NC, NS = info.num_cores, info.num_subcores
    NW = NC * NS
    N = tok_flat.shape[0]
    per_w = N // NW
    n_chunks = per_w // _ROWS_PER_DMA
    NV = D // 16
    mesh = plsc.VectorSubcoreMesh(core_axis_name="c", subcore_axis_name="s")

    @functools.partial(
        pl.kernel,
        out_type=jax.ShapeDtypeStruct((N, D), jnp.float32),
        mesh=mesh,
        compiler_params=pltpu.CompilerParams(needs_layout_passes=False),
        scratch_types=[
            pltpu.VMEM((per_w,), jnp.int32),
            pltpu.VMEM((S, D), jnp.float32),
            pltpu.VMEM((D,), jnp.float32),
            pltpu.VMEM((D,), jnp.float32),
            pltpu.VMEM((D,), jnp.float32),
            pltpu.VMEM((_ROWS_PER_DMA, D), jnp.float32),
            pltpu.VMEM((_ROWS_PER_DMA, D), jnp.float32),
            pltpu.VMEM((_ROWS_PER_DMA * 16,), jnp.float32),
            pltpu.VMEM((_ROWS_PER_DMA * 16,), jnp.float32),
            pltpu.SemaphoreType.DMA,
            pltpu.SemaphoreType.DMA,
            pltpu.SemaphoreType.DMA,
            pltpu.SemaphoreType.DMA,
            pltpu.SemaphoreType.DMA,
            pltpu.SemaphoreType.DMA,
        ],
    )
    def k(table_hbm, tok_hbm, seg_hbm, pe0_hbm, dte_hbm, gam_hbm, bet_hbm,
          out_hbm, idx_v, pe0_v, dte_v, gam_v, bet_v,
          rows0, rows1, segx0, segx1, sg0, sg1, so0, so1, ss0, ss1):
        wid = lax.axis_index("s") * NC + lax.axis_index("c")
        base = wid * per_w
        pltpu.sync_copy(tok_hbm.at[pl.ds(base, per_w)], idx_v)
        pltpu.sync_copy(pe0_hbm, pe0_v)
        pltpu.sync_copy(dte_hbm, dte_v)
        pltpu.sync_copy(gam_hbm, gam_v)
        pltpu.sync_copy(bet_hbm, bet_v)
        rows = (rows0, rows1)
        segx = (segx0, segx1)
        sgs = (sg0, sg1)
        sos = (so0, so1)
        sss = (ss0, ss1)
        dte = [dte_v[pl.ds(16 * i, 16)] for i in range(NV)]
        gam = [gam_v[pl.ds(16 * i, 16)] for i in range(NV)]
        bet = [bet_v[pl.ds(16 * i, 16)] for i in range(NV)]
        inv_d = jnp.float32(1.0 / D)

        def seg_src(j):
            return seg_hbm.at[pl.ds((base + j * _ROWS_PER_DMA) * 16,
                                    _ROWS_PER_DMA * 16)]

        pltpu.async_copy(
            table_hbm.at[idx_v.at[pl.ds(0, _ROWS_PER_DMA)]], rows0, sg0)
        pltpu.async_copy(seg_src(0), segx0, ss0)

        def do_chunk(j, b):
            rv, sg, so = rows[b], sgs[b], sos[b]
            sx, ss = segx[b], sss[b]
            b2 = 1 - b
            @pl.when(j + 1 < n_chunks)
            def _():
                @pl.when(j >= 1)
                def _():
                    pltpu.make_async_copy(
                        rows[b2],
                        out_hbm.at[pl.ds(base + (j - 1) * _ROWS_PER_DMA,
                                         _ROWS_PER_DMA)],
                        sos[b2]).wait()
                pltpu.async_copy(
                    table_hbm.at[idx_v.at[pl.ds((j + 1) * _ROWS_PER_DMA,
                                                _ROWS_PER_DMA)]],
                    rows[b2], sgs[b2])
                pltpu.async_copy(seg_src(j + 1), segx[b2], sss[b2])

            pltpu.make_async_copy(
                table_hbm.at[idx_v.at[pl.ds(j * _ROWS_PER_DMA,
                                            _ROWS_PER_DMA)]],
                rv, sg).wait()
            pltpu.make_async_copy(seg_src(j), sx, ss).wait()

            def one_token(t):
                tl = j * _ROWS_PER_DMA + t
                pos = lax.rem(tl, S)
                s_f = sx[pl.ds(t * 16, 16)]
                x = [rv[t, pl.ds(16 * i, 16)] + pe0_v[pos, pl.ds(16 * i, 16)]
                     + s_f * dte[i] for i in range(NV)]
                sv = (x[0] + x[1]) + (x[2] + x[3]) + (
                    (x[4] + x[5]) + (x[6] + x[7]))
                qv = (x[0] * x[0] + x[1] * x[1]) + (
                    x[2] * x[2] + x[3] * x[3]) + (
                    (x[4] * x[4] + x[5] * x[5]) + (x[6] * x[6] + x[7] * x[7]))
                Sv = jnp.full((16,), jnp.sum(sv))
                Qv = jnp.full((16,), jnp.sum(qv))
                mu = Sv * inv_d
                var = Qv * inv_d - mu * mu
                a = var + jnp.float32(_EPS)
                yi = jnp.int32(0x5F3759DF) - (
                    plsc.bitcast(a, jnp.int32) >> 1)
                y = plsc.bitcast(yi, jnp.float32)
                ah = a * jnp.float32(0.5)
                for _ in range(2):
                    y = y * (jnp.float32(1.5) - ah * y * y)
                return x, mu, y

            @plsc.parallel_loop(0, _ROWS_PER_DMA, unroll=2)
            def _(t):
                x, mu, y = one_token(t)
                for i in range(NV):
                    rv[t, pl.ds(16 * i, 16)] = (
                        (x[i] - mu) * y * gam[i] + bet[i])
            pltpu.async_copy(
                rv, out_hbm.at[pl.ds(base + j * _ROWS_PER_DMA,
                                     _ROWS_PER_DMA)], so)

        def chunk(j, _):
            for b in range(2):
                @pl.when(lax.rem(j, 2) == b)
                def _():
                    do_chunk(j, b)
            return 0

        lax.fori_loop(0, n_chunks, chunk, 0)
        for lastj in (n_chunks - 2, n_chunks - 1):
            pltpu.make_async_copy(
                rows[lastj % 2],
                out_hbm.at[pl.ds(base + lastj * _ROWS_PER_DMA,
                                 _ROWS_PER_DMA)],
                sos[lastj % 2]).wait()

    return k(word_emb, tok_flat, seg_flat, pe0, dte, gamma, beta)


def kernel(token_ids, segment_ids, word_emb, pos_emb, type_emb, gamma, beta):
    B, S = token_ids.shape
    D = word_emb.shape[1]
    N = B * S
    tok_flat = token_ids.reshape(N).astype(jnp.int32)
    gathered = _sc_gather(word_emb, tok_flat, rows_per_dma=_ROWS_PER_DMA)
    pe0 = pos_emb[:S] + type_emb[0][None, :]
    dte = (type_emb[1] - type_emb[0]).reshape(1, D)
    return _tc_ln(gathered.reshape(B, S, D), segment_ids.astype(jnp.int32),
                  pe0, dte, gamma.reshape(1, D),
                  beta.reshape(1, D), b_blk=64)


# speedup vs baseline: 1.0683x; 1.0683x over previous
import functools

import jax
import jax.numpy as jnp
from jax import lax
from jax.experimental import pallas as pl
from jax.experimental.pallas import tpu as pltpu
from jax.experimental.pallas import tpu_sc as plsc


_ROWS_PER_DMA = 128


def _sc_gather(word_emb, tok_flat, rows_per_dma):
    D = word_emb.shape[1]
    info = plsc.get_sparse_core_info()
    NC, NS = info.num_cores, info.num_subcores
    NW = NC * NS
    N = tok_flat.shape[0]
    per_w = N // NW
    n_chunks = per_w // rows_per_dma
    mesh = plsc.VectorSubcoreMesh(core_axis_name="c", subcore_axis_name="s")

    @functools.partial(
        pl.kernel,
        out_type=jax.ShapeDtypeStruct((N, D), jnp.float32),
        mesh=mesh,
        scratch_types=[
            pltpu.VMEM((per_w,), jnp.int32),
            pltpu.VMEM((rows_per_dma, D), jnp.float32),
            pltpu.VMEM((rows_per_dma, D), jnp.float32),
            pltpu.SemaphoreType.DMA,
            pltpu.SemaphoreType.DMA,
        ],
    )
    def k(table_hbm, idx_hbm, out_hbm, idx_v, rows0, rows1, sem0, sem1):
        wid = lax.axis_index("s") * NC + lax.axis_index("c")
        base = wid * per_w
        pltpu.sync_copy(idx_hbm.at[pl.ds(base, per_w)], idx_v)
        rows = (rows0, rows1)
        sems = (sem0, sem1)
        pltpu.async_copy(
            table_hbm.at[idx_v.at[pl.ds(0, rows_per_dma)]], rows0, sem0)

        def chunk(j, _):
            @pl.when(j + 1 < n_chunks)
            def _():
                for b in range(2):
                    @pl.when(lax.rem(j + 1, 2) == b)
                    def _():
                        pltpu.async_copy(
                            table_hbm.at[
                                idx_v.at[pl.ds((j + 1) * rows_per_dma,
                                               rows_per_dma)]],
                            rows[b], sems[b])

            for b in range(2):
                @pl.when(lax.rem(j, 2) == b)
                def _():
                    pltpu.make_async_copy(
                        table_hbm.at[
                            idx_v.at[pl.ds(j * rows_per_dma,
                                           rows_per_dma)]],
                        rows[b], sems[b]).wait()
                    pltpu.sync_copy(
                        rows[b],
                        out_hbm.at[pl.ds(base + j * rows_per_dma,
                                         rows_per_dma)])
            return 0

        lax.fori_loop(0, n_chunks, chunk, 0)

    return k(word_emb, tok_flat)


_EPS = 1e-12


def _ln_body(g_ref, seg_ref, pe_ref, te_ref, gamma_ref, beta_ref, o_ref):
    pe = pe_ref[...]
    te0 = te_ref[0, :]
    te1 = te_ref[1, :]
    seg = seg_ref[...]
    te = jnp.where(seg[..., None] == 0, te0[None, None, :], te1[None, None, :])
    x = g_ref[...] + te + pe[None, :, :]
    mu = jnp.mean(x, axis=-1, keepdims=True)
    xc = x - mu
    var = jnp.mean(xc * xc, axis=-1, keepdims=True)
    normed = xc * lax.rsqrt(var + _EPS)
    o_ref[...] = normed * gamma_ref[0, :] + beta_ref[0, :]


def _tc_ln(gathered, segment_ids, pe, te, gamma, beta, b_blk):
    B, S, D = gathered.shape
    grid = (B // b_blk,)
    return pl.pallas_call(
        _ln_body,
        grid=grid,
        in_specs=[
            pl.BlockSpec((b_blk, S, D), lambda i: (i, 0, 0)),
            pl.BlockSpec((b_blk, S), lambda i: (i, 0)),
            pl.BlockSpec((S, D), lambda i: (0, 0)),
            pl.BlockSpec((2, D), lambda i: (0, 0)),
            pl.BlockSpec((1, D), lambda i: (0, 0)),
            pl.BlockSpec((1, D), lambda i: (0, 0)),
        ],
        out_specs=pl.BlockSpec((b_blk, S, D), lambda i: (i, 0, 0)),
        out_shape=jax.ShapeDtypeStruct((B, S, D), jnp.float32),
    )(gathered, segment_ids, pe, te, gamma, beta)


def kernel(token_ids, segment_ids, word_emb, pos_emb, type_emb, gamma, beta):
    B, S = token_ids.shape
    D = word_emb.shape[1]
    N = B * S
    tok_flat = token_ids.reshape(N).astype(jnp.int32)
    gathered = _sc_gather(word_emb, tok_flat, rows_per_dma=_ROWS_PER_DMA)
    return _tc_ln(gathered.reshape(B, S, D), segment_ids.astype(jnp.int32),
                  pos_emb[:S], type_emb, gamma.reshape(1, D),
                  beta.reshape(1, D), b_blk=64)
